# Initial kernel scaffold; baseline (speedup 1.0000x reference)
import jax
import jax.numpy as jnp
from jax.experimental import pallas as pl


def kernel(x, edge_index, W1, b1, W2, b2, W3, b3):
    raise NotImplementedError("write your pallas kernel here")


# speedup vs baseline: 7.4024x; 7.4024x over previous
import functools

import jax
import jax.numpy as jnp
from jax import lax
from jax.experimental import pallas as pl
from jax.experimental.pallas import tpu as pltpu
from jax.experimental.pallas import tpu_sc as plsc

N = 10000
NP = 10240
E = 320000
D = 128
NC = 2
NS = 16
NW = NC * NS
EPT = E // NW
B = 80
KB = EPT // B
STRIPE = NP // NS

_mesh = plsc.VectorSubcoreMesh(core_axis_name="c", subcore_axis_name="s")


@functools.partial(
    pl.kernel,
    mesh=_mesh,
    out_type=(
        jax.ShapeDtypeStruct((NP, D), jnp.float32),
        jax.ShapeDtypeStruct((NP, D), jnp.float32),
        jax.ShapeDtypeStruct((NP,), jnp.float32),
        jax.ShapeDtypeStruct((NP,), jnp.float32),
    ),
    scratch_types=(
        pltpu.VMEM((KB, B), jnp.int32),
        pltpu.VMEM((KB, B), jnp.int32),
        pltpu.VMEM((B, D), jnp.float32),
        pltpu.VMEM((B,), jnp.float32),
        pltpu.VMEM_SHARED((NP, D), jnp.float32),
        pltpu.VMEM_SHARED((NP,), jnp.float32),
        pltpu.SemaphoreType.DMA,
    ),
)
def _sc_agg_deg(g_hbm, srcr_hbm, dstr_hbm, z2d_hbm, z1d_hbm, ones_hbm,
                part0_hbm, part1_hbm, deg0_hbm, deg1_hbm,
                src_v, dst_v, rows_v, ones_v, acc_sh, deg_sh, sem):
    c = lax.axis_index("c")
    s = lax.axis_index("s")
    w = c * NS + s
    pltpu.sync_copy(srcr_hbm.at[w], src_v)
    pltpu.sync_copy(dstr_hbm.at[w], dst_v)
    pltpu.sync_copy(ones_hbm, ones_v)
    pltpu.sync_copy(z2d_hbm, acc_sh.at[pl.ds(s * STRIPE, STRIPE)])
    pltpu.sync_copy(z1d_hbm, deg_sh.at[pl.ds(s * STRIPE, STRIPE)])
    plsc.subcore_barrier()

    def step(k, carry):
        pltpu.async_copy(g_hbm.at[src_v.at[k]], rows_v, sem).wait()
        pltpu.sync_copy(rows_v, acc_sh.at[dst_v.at[k]], add=True)
        pltpu.sync_copy(ones_v, deg_sh.at[dst_v.at[k]], add=True)
        return carry

    lax.fori_loop(0, KB, step, 0)
    plsc.subcore_barrier()
    sl = pl.ds(s * STRIPE, STRIPE)

    @pl.when(c == 0)
    def _():
        pltpu.sync_copy(acc_sh.at[sl], part0_hbm.at[sl])
        pltpu.sync_copy(deg_sh.at[sl], deg0_hbm.at[sl])

    @pl.when(c == 1)
    def _():
        pltpu.sync_copy(acc_sh.at[sl], part1_hbm.at[sl])
        pltpu.sync_copy(deg_sh.at[sl], deg1_hbm.at[sl])


@functools.partial(
    pl.kernel,
    mesh=_mesh,
    out_type=(
        jax.ShapeDtypeStruct((NP, D), jnp.float32),
        jax.ShapeDtypeStruct((NP, D), jnp.float32),
    ),
    scratch_types=(
        pltpu.VMEM((KB, B), jnp.int32),
        pltpu.VMEM((KB, B), jnp.int32),
        pltpu.VMEM((B, D), jnp.float32),
        pltpu.VMEM_SHARED((NP, D), jnp.float32),
        pltpu.SemaphoreType.DMA,
    ),
)
def _sc_agg(g_hbm, srcr_hbm, dstr_hbm, z2d_hbm,
            part0_hbm, part1_hbm,
            src_v, dst_v, rows_v, acc_sh, sem):
    c = lax.axis_index("c")
    s = lax.axis_index("s")
    w = c * NS + s
    pltpu.sync_copy(srcr_hbm.at[w], src_v)
    pltpu.sync_copy(dstr_hbm.at[w], dst_v)
    pltpu.sync_copy(z2d_hbm, acc_sh.at[pl.ds(s * STRIPE, STRIPE)])
    plsc.subcore_barrier()

    def step(k, carry):
        pltpu.async_copy(g_hbm.at[src_v.at[k]], rows_v, sem).wait()
        pltpu.sync_copy(rows_v, acc_sh.at[dst_v.at[k]], add=True)
        return carry

    lax.fori_loop(0, KB, step, 0)
    plsc.subcore_barrier()
    sl = pl.ds(s * STRIPE, STRIPE)

    @pl.when(c == 0)
    def _():
        pltpu.sync_copy(acc_sh.at[sl], part0_hbm.at[sl])

    @pl.when(c == 1)
    def _():
        pltpu.sync_copy(acc_sh.at[sl], part1_hbm.at[sl])


RB = 1000


def _mm_body(x_ref, w_ref, o_ref):
    o_ref[...] = jnp.dot(x_ref[...], w_ref[...],
                         preferred_element_type=jnp.float32)


def _tc_matmul(x, W):
    return pl.pallas_call(
        _mm_body,
        grid=(N // RB,),
        in_specs=[pl.BlockSpec((RB, D), lambda i: (i, 0)),
                  pl.BlockSpec((D, D), lambda i: (0, 0))],
        out_specs=pl.BlockSpec((RB, D), lambda i: (i, 0)),
        out_shape=jax.ShapeDtypeStruct((N, D), jnp.float32),
    )(x, W)


def _fused_body(g_ref, p0_ref, p1_ref, d0_ref, d1_ref, b_ref, w_ref, o_ref):
    inv = 1.0 / jnp.maximum(d0_ref[...] + d1_ref[...], 1.0)
    h = g_ref[...] + b_ref[...] + (p0_ref[...] + p1_ref[...]) * inv
    h = jnp.maximum(h, 0.0)
    o_ref[...] = jnp.dot(h, w_ref[...], preferred_element_type=jnp.float32)


def _tc_fused(g, p0, p1, d0, d1, b, Wn):
    return pl.pallas_call(
        _fused_body,
        grid=(N // RB,),
        in_specs=[pl.BlockSpec((RB, D), lambda i: (i, 0)),
                  pl.BlockSpec((RB, D), lambda i: (i, 0)),
                  pl.BlockSpec((RB, D), lambda i: (i, 0)),
                  pl.BlockSpec((RB, 1), lambda i: (i, 0)),
                  pl.BlockSpec((RB, 1), lambda i: (i, 0)),
                  pl.BlockSpec((1, D), lambda i: (0, 0)),
                  pl.BlockSpec((D, D), lambda i: (0, 0))],
        out_specs=pl.BlockSpec((RB, D), lambda i: (i, 0)),
        out_shape=jax.ShapeDtypeStruct((N, D), jnp.float32),
    )(g, p0, p1, d0, d1, b, Wn)


def _final_body(g_ref, p0_ref, p1_ref, d0_ref, d1_ref, b_ref, o_ref):
    inv = 1.0 / jnp.maximum(d0_ref[...] + d1_ref[...], 1.0)
    o_ref[...] = (g_ref[...] + b_ref[...]
                  + (p0_ref[...] + p1_ref[...]) * inv)


def _tc_final(g, p0, p1, d0, d1, b):
    return pl.pallas_call(
        _final_body,
        grid=(N // RB,),
        in_specs=[pl.BlockSpec((RB, D), lambda i: (i, 0)),
                  pl.BlockSpec((RB, D), lambda i: (i, 0)),
                  pl.BlockSpec((RB, D), lambda i: (i, 0)),
                  pl.BlockSpec((RB, 1), lambda i: (i, 0)),
                  pl.BlockSpec((RB, 1), lambda i: (i, 0)),
                  pl.BlockSpec((1, D), lambda i: (0, 0))],
        out_specs=pl.BlockSpec((RB, D), lambda i: (i, 0)),
        out_shape=jax.ShapeDtypeStruct((N, D), jnp.float32),
    )(g, p0, p1, d0, d1, b)


def kernel(x, edge_index, W1, b1, W2, b2, W3, b3):
    src = edge_index[0].astype(jnp.int32).reshape(NW, KB, B)
    dst = edge_index[1].astype(jnp.int32).reshape(NW, KB, B)
    z2d = jnp.zeros((STRIPE, D), jnp.float32)
    z1d = jnp.zeros((STRIPE,), jnp.float32)
    ones = jnp.ones((B,), jnp.float32)
    b1r = b1.reshape(1, D)
    b2r = b2.reshape(1, D)
    b3r = b3.reshape(1, D)

    g1 = _tc_matmul(x, W1)
    q0, q1, dg0, dg1 = _sc_agg_deg(g1, src, dst, z2d, z1d, ones)
    d0 = dg0.reshape(NP, 1)
    d1 = dg1.reshape(NP, 1)
    g2 = _tc_fused(g1, q0, q1, d0, d1, b1r, W2)
    r0, r1 = _sc_agg(g2, src, dst, z2d)
    g3 = _tc_fused(g2, r0, r1, d0, d1, b2r, W3)
    t0, t1 = _sc_agg(g3, src, dst, z2d)
    return _tc_final(g3, t0, t1, d0, d1, b3r)


# speedup vs baseline: 7.9442x; 1.0732x over previous
import functools

import jax
import jax.numpy as jnp
from jax import lax
from jax.experimental import pallas as pl
from jax.experimental.pallas import tpu as pltpu
from jax.experimental.pallas import tpu_sc as plsc

N = 10000
NP = 10112
NPD = 10240
E = 320000
EPAD = 327680
D = 128
HD = 64
NC = 2
NS = 16
B = 128
KB = EPAD // (NS * B)
KSS = KB // 2
STRIPE = NP // NS
DSTRIPE = NPD // NS

_mesh = plsc.VectorSubcoreMesh(core_axis_name="c", subcore_axis_name="s")


def _sc_body(with_deg):
    def body(*args):
        if with_deg:
            (g_hbm, srcr_hbm, dstr_hbm, z2d_hbm, z1d_hbm, ones_hbm,
             part_hbm, deg_hbm,
             src_v, dst_v, rows_v, ones_v, acc_sh, deg_sh,
             g0, g1, s0, s1, dsem) = args
        else:
            (g_hbm, srcr_hbm, dstr_hbm, z2d_hbm,
             part_hbm,
             src_v, dst_v, rows_v, acc_sh,
             g0, g1, s0, s1) = args
        gsems = (g0, g1)
        ssems = (s0, s1)
        c = lax.axis_index("c")
        s = lax.axis_index("s")
        pltpu.sync_copy(srcr_hbm.at[s], src_v)
        pltpu.sync_copy(dstr_hbm.at[s], dst_v)
        if with_deg:
            pltpu.sync_copy(ones_hbm, ones_v)

            @pl.when(c == 0)
            def _():
                pltpu.sync_copy(z1d_hbm, deg_sh.at[pl.ds(s * DSTRIPE, DSTRIPE)])

        pltpu.sync_copy(z2d_hbm, acc_sh.at[pl.ds(s * STRIPE, STRIPE)])
        plsc.subcore_barrier()
        gsrc = g_hbm.at[c]

        def fire_g(k, j):
            pltpu.async_copy(gsrc.at[src_v.at[k]], rows_v.at[j], gsems[j])

        def wait_g(k, j):
            pltpu.make_async_copy(gsrc.at[src_v.at[k]],
                                  rows_v.at[j], gsems[j]).wait()

        def fire_s(k, j):
            pltpu.async_copy(rows_v.at[j], acc_sh.at[dst_v.at[k]],
                             ssems[j], add=True)
            if with_deg:
                @pl.when(c == 0)
                def _():
                    pltpu.async_copy(ones_v, deg_sh.at[dst_v.at[k]],
                                     dsem, add=True)

        def wait_s(k, j):
            pltpu.make_async_copy(rows_v.at[j], acc_sh.at[dst_v.at[k]],
                                  ssems[j]).wait()

        fire_g(0, 0)

        def m_body(m, carry):
            k0 = 2 * m
            k1 = k0 + 1
            wait_g(k0, 0)

            @pl.when(m > 0)
            def _():
                wait_s(k1 - 2, 1)

            fire_g(k1, 1)
            fire_s(k0, 0)
            wait_g(k1, 1)

            @pl.when(m < KSS - 1)
            def _():
                wait_s(k0, 0)
                fire_g(k0 + 2, 0)

            fire_s(k1, 1)
            return carry

        lax.fori_loop(0, KSS, m_body, 0)
        wait_s(KB - 2, 0)
        wait_s(KB - 1, 1)
        if with_deg:
            @pl.when(c == 0)
            def _():
                def d_body(k, carry):
                    pltpu.make_async_copy(ones_v, deg_sh.at[dst_v.at[k]],
                                          dsem).wait()
                    return carry
                lax.fori_loop(0, KB, d_body, 0)

        plsc.subcore_barrier()
        sl = pl.ds(s * STRIPE, STRIPE)
        pltpu.sync_copy(acc_sh.at[sl], part_hbm.at[c, sl])
        if with_deg:
            dsl = pl.ds(s * DSTRIPE, DSTRIPE)

            @pl.when(c == 0)
            def _():
                pltpu.sync_copy(deg_sh.at[dsl], deg_hbm.at[dsl])

    return body


_sc_params = pltpu.CompilerParams(use_tc_tiling_on_sc=False)

_sc_agg_deg = functools.partial(
    pl.kernel,
    mesh=_mesh,
    compiler_params=_sc_params,
    out_type=(
        jax.ShapeDtypeStruct((NC, NP, HD), jnp.float32),
        jax.ShapeDtypeStruct((NPD,), jnp.float32),
    ),
    scratch_types=(
        pltpu.VMEM((KB, B), jnp.int32),
        pltpu.VMEM((KB, B), jnp.int32),
        pltpu.VMEM((2, B, HD), jnp.float32),
        pltpu.VMEM((B,), jnp.float32),
        pltpu.VMEM_SHARED((NP, HD), jnp.float32),
        pltpu.VMEM_SHARED((NPD,), jnp.float32),
        pltpu.SemaphoreType.DMA,
        pltpu.SemaphoreType.DMA,
        pltpu.SemaphoreType.DMA,
        pltpu.SemaphoreType.DMA,
        pltpu.SemaphoreType.DMA,
    ),
)(_sc_body(True))


_sc_agg = functools.partial(
    pl.kernel,
    mesh=_mesh,
    compiler_params=_sc_params,
    out_type=jax.ShapeDtypeStruct((NC, NP, HD), jnp.float32),
    scratch_types=(
        pltpu.VMEM((KB, B), jnp.int32),
        pltpu.VMEM((KB, B), jnp.int32),
        pltpu.VMEM((2, B, HD), jnp.float32),
        pltpu.VMEM_SHARED((NP, HD), jnp.float32),
        pltpu.SemaphoreType.DMA,
        pltpu.SemaphoreType.DMA,
        pltpu.SemaphoreType.DMA,
        pltpu.SemaphoreType.DMA,
    ),
)(_sc_body(False))


RB = 1000


def _mm_body(x_ref, w_ref, o_ref):
    o_ref[0] = jnp.dot(x_ref[...], w_ref[0],
                       preferred_element_type=jnp.float32)


def _tc_matmul(x, Wr):
    return pl.pallas_call(
        _mm_body,
        grid=(NC, N // RB),
        in_specs=[pl.BlockSpec((RB, D), lambda c, i: (i, 0)),
                  pl.BlockSpec((1, D, HD), lambda c, i: (c, 0, 0))],
        out_specs=pl.BlockSpec((1, RB, HD), lambda c, i: (c, i, 0)),
        out_shape=jax.ShapeDtypeStruct((NC, NP, HD), jnp.float32),
    )(x, Wr)


def _fused_body(g0_ref, g1_ref, p0_ref, p1_ref, d_ref, b0_ref, b1_ref,
                w_ref, o_ref):
    inv = 1.0 / jnp.maximum(d_ref[...], 1.0)
    h0 = jnp.maximum(g0_ref[0] + b0_ref[0] + p0_ref[0] * inv, 0.0)
    h1 = jnp.maximum(g1_ref[0] + b1_ref[0] + p1_ref[0] * inv, 0.0)
    o_ref[0] = (jnp.dot(h0, w_ref[0, :HD, :],
                        preferred_element_type=jnp.float32)
                + jnp.dot(h1, w_ref[0, HD:, :],
                          preferred_element_type=jnp.float32))


def _tc_fused(g, p, d, br, Wr):
    return pl.pallas_call(
        _fused_body,
        grid=(NC, N // RB),
        in_specs=[pl.BlockSpec((1, RB, HD), lambda c, i: (0, i, 0)),
                  pl.BlockSpec((1, RB, HD), lambda c, i: (1, i, 0)),
                  pl.BlockSpec((1, RB, HD), lambda c, i: (0, i, 0)),
                  pl.BlockSpec((1, RB, HD), lambda c, i: (1, i, 0)),
                  pl.BlockSpec((RB, 1), lambda c, i: (i, 0)),
                  pl.BlockSpec((1, 1, HD), lambda c, i: (0, 0, 0)),
                  pl.BlockSpec((1, 1, HD), lambda c, i: (1, 0, 0)),
                  pl.BlockSpec((1, D, HD), lambda c, i: (c, 0, 0))],
        out_specs=pl.BlockSpec((1, RB, HD), lambda c, i: (c, i, 0)),
        out_shape=jax.ShapeDtypeStruct((NC, NP, HD), jnp.float32),
    )(g, g, p, p, d, br, br, Wr)


def _final_body(g0_ref, g1_ref, p0_ref, p1_ref, d_ref, b0_ref, b1_ref, o_ref):
    inv = 1.0 / jnp.maximum(d_ref[...], 1.0)
    o_ref[:, :HD] = g0_ref[0] + b0_ref[0] + p0_ref[0] * inv
    o_ref[:, HD:] = g1_ref[0] + b1_ref[0] + p1_ref[0] * inv


def _tc_final(g, p, d, br):
    return pl.pallas_call(
        _final_body,
        grid=(N // RB,),
        in_specs=[pl.BlockSpec((1, RB, HD), lambda i: (0, i, 0)),
                  pl.BlockSpec((1, RB, HD), lambda i: (1, i, 0)),
                  pl.BlockSpec((1, RB, HD), lambda i: (0, i, 0)),
                  pl.BlockSpec((1, RB, HD), lambda i: (1, i, 0)),
                  pl.BlockSpec((RB, 1), lambda i: (i, 0)),
                  pl.BlockSpec((1, 1, HD), lambda i: (0, 0, 0)),
                  pl.BlockSpec((1, 1, HD), lambda i: (1, 0, 0))],
        out_specs=pl.BlockSpec((RB, D), lambda i: (i, 0)),
        out_shape=jax.ShapeDtypeStruct((N, D), jnp.float32),
    )(g, g, p, p, d, br, br)


def kernel(x, edge_index, W1, b1, W2, b2, W3, b3):
    npad = EPAD - E
    srcf = edge_index[0].astype(jnp.int32)
    dstf = edge_index[1].astype(jnp.int32)
    ar = jnp.arange(npad, dtype=jnp.int32)
    pad_src = (ar * 131) % N
    pad_dst = N + (ar % (NP - N))
    srcp = jnp.concatenate([srcf, pad_src]).reshape(NS, KB, B)
    dstp = jnp.concatenate([dstf, pad_dst]).reshape(NS, KB, B)
    z2d = jnp.zeros((STRIPE, HD), jnp.float32)
    z1d = jnp.zeros((DSTRIPE,), jnp.float32)
    ones = jnp.ones((B,), jnp.float32)
    W1r = jnp.stack((W1[:, :HD], W1[:, HD:]))
    W2r = jnp.stack((W2[:, :HD], W2[:, HD:]))
    W3r = jnp.stack((W3[:, :HD], W3[:, HD:]))
    b1r = b1.reshape(NC, 1, HD)
    b2r = b2.reshape(NC, 1, HD)
    b3r = b3.reshape(NC, 1, HD)

    g1 = _tc_matmul(x, W1r)
    p1, dg = _sc_agg_deg(g1, srcp, dstp, z2d, z1d, ones)
    d = dg.reshape(NPD, 1)
    g2 = _tc_fused(g1, p1, d, b1r, W2r)
    p2 = _sc_agg(g2, srcp, dstp, z2d)
    g3 = _tc_fused(g2, p2, d, b2r, W3r)
    p3 = _sc_agg(g3, srcp, dstp, z2d)
    return _tc_final(g3, p3, d, b3r)


# speedup vs baseline: 9.6924x; 1.2201x over previous
import functools

import jax
import jax.numpy as jnp
from jax import lax
from jax.experimental import pallas as pl
from jax.experimental.pallas import tpu as pltpu
from jax.experimental.pallas import tpu_sc as plsc

N = 10000
NP = 10112
NPD = 10240
E = 320000
EPAD = 327680
D = 128
HD = 64
NC = 2
NS = 16
B = 128
KB = EPAD // (NS * B)
KSS3 = (KB - 1) // 3
NWK = NC * NS
KBD = EPAD // (NWK * B)
STRIPE = NP // NS
DSTRIPE = NPD // NS

_mesh = plsc.VectorSubcoreMesh(core_axis_name="c", subcore_axis_name="s")


def _sc_deg_body(dstr_hbm, z1d_hbm, ones_hbm, deg0_hbm, deg1_hbm,
                 dst_v, ones_v, deg_sh, dsem):
    c = lax.axis_index("c")
    s = lax.axis_index("s")
    w = c * NS + s
    pltpu.sync_copy(dstr_hbm.at[w], dst_v)
    pltpu.sync_copy(ones_hbm, ones_v)
    pltpu.sync_copy(z1d_hbm, deg_sh.at[pl.ds(s * DSTRIPE, DSTRIPE)])
    plsc.subcore_barrier()

    def f_body(k, carry):
        pltpu.async_copy(ones_v, deg_sh.at[dst_v.at[k]], dsem, add=True)
        return carry

    lax.fori_loop(0, KBD, f_body, 0)

    def d_body(k, carry):
        pltpu.make_async_copy(ones_v, deg_sh.at[dst_v.at[k]], dsem).wait()
        return carry

    lax.fori_loop(0, KBD, d_body, 0)
    plsc.subcore_barrier()
    dsl = pl.ds(s * DSTRIPE, DSTRIPE)

    @pl.when(c == 0)
    def _():
        pltpu.sync_copy(deg_sh.at[dsl], deg0_hbm.at[dsl])

    @pl.when(c == 1)
    def _():
        pltpu.sync_copy(deg_sh.at[dsl], deg1_hbm.at[dsl])


def _sc_agg_body(g_hbm, srcr_hbm, dstr_hbm, z2d_hbm,
                 part_hbm,
                 src_v, dst_v, rows_v, acc_sh,
                 g0, g1, g2, s0, s1, s2):
    gsems = (g0, g1, g2)
    ssems = (s0, s1, s2)
    c = lax.axis_index("c")
    s = lax.axis_index("s")
    pltpu.sync_copy(srcr_hbm.at[s], src_v)
    pltpu.sync_copy(dstr_hbm.at[s], dst_v)
    pltpu.sync_copy(z2d_hbm, acc_sh.at[pl.ds(s * STRIPE, STRIPE)])
    plsc.subcore_barrier()
    gsrc = g_hbm.at[c]

    def fire_g(k, j):
        pltpu.async_copy(gsrc.at[src_v.at[k]], rows_v.at[j], gsems[j])

    def wait_g(k, j):
        pltpu.make_async_copy(gsrc.at[src_v.at[k]],
                              rows_v.at[j], gsems[j]).wait()

    def fire_s(k, j):
        pltpu.async_copy(rows_v.at[j], acc_sh.at[dst_v.at[k]],
                         ssems[j], add=True)

    def wait_s(k, j):
        pltpu.make_async_copy(rows_v.at[j], acc_sh.at[dst_v.at[k]],
                              ssems[j]).wait()

    fire_g(0, 0)
    fire_g(1, 1)

    def m_body(m, carry):
        k = 3 * m

        @pl.when(m > 0)
        def _():
            wait_s(k - 1, 2)

        fire_g(k + 2, 2)
        wait_g(k, 0)
        fire_s(k, 0)
        wait_g(k + 1, 1)
        fire_s(k + 1, 1)
        wait_s(k, 0)
        fire_g(k + 3, 0)
        wait_g(k + 2, 2)
        fire_s(k + 2, 2)

        @pl.when(m < KSS3 - 1)
        def _():
            wait_s(k + 1, 1)
            fire_g(k + 4, 1)

        return carry

    lax.fori_loop(0, KSS3, m_body, 0)
    wait_g(KB - 1, 0)
    fire_s(KB - 1, 0)
    wait_s(KB - 3, 1)
    wait_s(KB - 2, 2)
    wait_s(KB - 1, 0)

    plsc.subcore_barrier()
    sl = pl.ds(s * STRIPE, STRIPE)
    pltpu.sync_copy(acc_sh.at[sl], part_hbm.at[c, sl])


_sc_params = pltpu.CompilerParams(use_tc_tiling_on_sc=False)

_sc_deg = functools.partial(
    pl.kernel,
    mesh=_mesh,
    compiler_params=_sc_params,
    out_type=(
        jax.ShapeDtypeStruct((NPD,), jnp.float32),
        jax.ShapeDtypeStruct((NPD,), jnp.float32),
    ),
    scratch_types=(
        pltpu.VMEM((KBD, B), jnp.int32),
        pltpu.VMEM((B,), jnp.float32),
        pltpu.VMEM_SHARED((NPD,), jnp.float32),
        pltpu.SemaphoreType.DMA,
    ),
)(_sc_deg_body)


_sc_agg = functools.partial(
    pl.kernel,
    mesh=_mesh,
    compiler_params=_sc_params,
    out_type=jax.ShapeDtypeStruct((NC, NP, HD), jnp.float32),
    scratch_types=(
        pltpu.VMEM((KB, B), jnp.int32),
        pltpu.VMEM((KB, B), jnp.int32),
        pltpu.VMEM((3, B, HD), jnp.float32),
        pltpu.VMEM_SHARED((NP, HD), jnp.float32),
        pltpu.SemaphoreType.DMA,
        pltpu.SemaphoreType.DMA,
        pltpu.SemaphoreType.DMA,
        pltpu.SemaphoreType.DMA,
        pltpu.SemaphoreType.DMA,
        pltpu.SemaphoreType.DMA,
    ),
)(_sc_agg_body)


RB = 1000


def _mm_body(x_ref, w_ref, o_ref):
    o_ref[0] = jnp.dot(x_ref[...], w_ref[0],
                       preferred_element_type=jnp.float32)


def _tc_matmul(x, Wr):
    return pl.pallas_call(
        _mm_body,
        grid=(NC, N // RB),
        in_specs=[pl.BlockSpec((RB, D), lambda c, i: (i, 0)),
                  pl.BlockSpec((1, D, HD), lambda c, i: (c, 0, 0))],
        out_specs=pl.BlockSpec((1, RB, HD), lambda c, i: (c, i, 0)),
        out_shape=jax.ShapeDtypeStruct((NC, NP, HD), jnp.float32),
    )(x, Wr)


def _fused_body(g0_ref, g1_ref, p0_ref, p1_ref, d0_ref, d1_ref, b0_ref,
                b1_ref, w_ref, o_ref):
    inv = 1.0 / jnp.maximum(d0_ref[...] + d1_ref[...], 1.0)
    h0 = jnp.maximum(g0_ref[0] + b0_ref[0] + p0_ref[0] * inv, 0.0)
    h1 = jnp.maximum(g1_ref[0] + b1_ref[0] + p1_ref[0] * inv, 0.0)
    o_ref[0] = (jnp.dot(h0, w_ref[0, :HD, :],
                        preferred_element_type=jnp.float32)
                + jnp.dot(h1, w_ref[0, HD:, :],
                          preferred_element_type=jnp.float32))


def _tc_fused(g, p, d0, d1, br, Wr):
    return pl.pallas_call(
        _fused_body,
        grid=(NC, N // RB),
        in_specs=[pl.BlockSpec((1, RB, HD), lambda c, i: (0, i, 0)),
                  pl.BlockSpec((1, RB, HD), lambda c, i: (1, i, 0)),
                  pl.BlockSpec((1, RB, HD), lambda c, i: (0, i, 0)),
                  pl.BlockSpec((1, RB, HD), lambda c, i: (1, i, 0)),
                  pl.BlockSpec((RB, 1), lambda c, i: (i, 0)),
                  pl.BlockSpec((RB, 1), lambda c, i: (i, 0)),
                  pl.BlockSpec((1, 1, HD), lambda c, i: (0, 0, 0)),
                  pl.BlockSpec((1, 1, HD), lambda c, i: (1, 0, 0)),
                  pl.BlockSpec((1, D, HD), lambda c, i: (c, 0, 0))],
        out_specs=pl.BlockSpec((1, RB, HD), lambda c, i: (c, i, 0)),
        out_shape=jax.ShapeDtypeStruct((NC, NP, HD), jnp.float32),
    )(g, g, p, p, d0, d1, br, br, Wr)


def _final_body(g0_ref, g1_ref, p0_ref, p1_ref, d0_ref, d1_ref, b0_ref,
                b1_ref, o_ref):
    inv = 1.0 / jnp.maximum(d0_ref[...] + d1_ref[...], 1.0)
    o_ref[:, :HD] = g0_ref[0] + b0_ref[0] + p0_ref[0] * inv
    o_ref[:, HD:] = g1_ref[0] + b1_ref[0] + p1_ref[0] * inv


def _tc_final(g, p, d0, d1, br):
    return pl.pallas_call(
        _final_body,
        grid=(N // RB,),
        in_specs=[pl.BlockSpec((1, RB, HD), lambda i: (0, i, 0)),
                  pl.BlockSpec((1, RB, HD), lambda i: (1, i, 0)),
                  pl.BlockSpec((1, RB, HD), lambda i: (0, i, 0)),
                  pl.BlockSpec((1, RB, HD), lambda i: (1, i, 0)),
                  pl.BlockSpec((RB, 1), lambda i: (i, 0)),
                  pl.BlockSpec((RB, 1), lambda i: (i, 0)),
                  pl.BlockSpec((1, 1, HD), lambda i: (0, 0, 0)),
                  pl.BlockSpec((1, 1, HD), lambda i: (1, 0, 0))],
        out_specs=pl.BlockSpec((RB, D), lambda i: (i, 0)),
        out_shape=jax.ShapeDtypeStruct((N, D), jnp.float32),
    )(g, g, p, p, d0, d1, br, br)


def kernel(x, edge_index, W1, b1, W2, b2, W3, b3):
    npad = EPAD - E
    srcf = edge_index[0].astype(jnp.int32)
    dstf = edge_index[1].astype(jnp.int32)
    ar = jnp.arange(npad, dtype=jnp.int32)
    pad_src = (ar * 131) % N
    pad_dst = N + (ar % (NP - N))
    srcp = jnp.concatenate([srcf, pad_src]).reshape(NS, KB, B)
    dstp = jnp.concatenate([dstf, pad_dst]).reshape(NS, KB, B)
    z2d = jnp.zeros((STRIPE, HD), jnp.float32)
    z1d = jnp.zeros((DSTRIPE,), jnp.float32)
    ones = jnp.ones((B,), jnp.float32)
    W1r = jnp.stack((W1[:, :HD], W1[:, HD:]))
    W2r = jnp.stack((W2[:, :HD], W2[:, HD:]))
    W3r = jnp.stack((W3[:, :HD], W3[:, HD:]))
    b1r = b1.reshape(NC, 1, HD)
    b2r = b2.reshape(NC, 1, HD)
    b3r = b3.reshape(NC, 1, HD)

    dstp2 = dstp.reshape(NWK, KBD, B)
    dg0, dg1 = _sc_deg(dstp2, z1d, ones)
    g1 = _tc_matmul(x, W1r)
    p1 = _sc_agg(g1, srcp, dstp, z2d)
    d0 = dg0.reshape(NPD, 1)
    d1 = dg1.reshape(NPD, 1)
    g2 = _tc_fused(g1, p1, d0, d1, b1r, W2r)
    p2 = _sc_agg(g2, srcp, dstp, z2d)
    g3 = _tc_fused(g2, p2, d0, d1, b2r, W3r)
    p3 = _sc_agg(g3, srcp, dstp, z2d)
    return _tc_final(g3, p3, d0, d1, b3r)


# speedup vs baseline: 10.6997x; 1.1039x over previous
import functools

import jax
import jax.numpy as jnp
from jax import lax
from jax.experimental import pallas as pl
from jax.experimental.pallas import tpu as pltpu
from jax.experimental.pallas import tpu_sc as plsc

N = 10000
NP = 10112
NPD = 10240
E = 320000
EPAD = 327680
D = 128
HD = 64
NC = 2
NS = 16
B = 128
KB = EPAD // (NS * B)
SEG = 32
NSEG = KB // SEG
STRIPE = NP // NS
DSTRIPE = NPD // NS

_mesh = plsc.VectorSubcoreMesh(core_axis_name="c", subcore_axis_name="s")


def _sc_agg_body(with_deg):
    def body(*args):
        if with_deg:
            (g_hbm, srcr_hbm, dstr_hbm, z2d_hbm, z1d_hbm, ones_hbm,
             part_hbm, deg0_hbm, deg1_hbm,
             srcseg, dstseg, rows_v, ones_v, acc_sh, deg_sh,
             g0, g1, g2, g3, s0, s1, s2, s3, t0, t1, dsem) = args
        else:
            (g_hbm, srcr_hbm, dstr_hbm, z2d_hbm,
             part_hbm,
             srcseg, dstseg, rows_v, acc_sh,
             g0, g1, g2, g3, s0, s1, s2, s3, t0, t1) = args
        gsems = (g0, g1, g2, g3)
        ssems = (s0, s1, s2, s3)
        stsems = (t0, t1)
        c = lax.axis_index("c")
        s = lax.axis_index("s")
        pltpu.sync_copy(srcr_hbm.at[s, 0], srcseg.at[0])
        pltpu.sync_copy(dstr_hbm.at[s, 0], dstseg.at[0])
        if with_deg:
            pltpu.sync_copy(ones_hbm, ones_v)

            @pl.when(c == 0)
            def _():
                pltpu.sync_copy(z1d_hbm, deg_sh.at[pl.ds(s * DSTRIPE, DSTRIPE)])

        pltpu.sync_copy(z2d_hbm, acc_sh.at[pl.ds(s * STRIPE, STRIPE)])
        plsc.subcore_barrier()
        gsrc = g_hbm.at[c]

        def fire_stage(t1_, slot):
            pltpu.async_copy(srcr_hbm.at[s, t1_], srcseg.at[slot],
                             stsems[slot])
            pltpu.async_copy(dstr_hbm.at[s, t1_], dstseg.at[slot],
                             stsems[slot])

        def wait_stage(t1_, slot):
            pltpu.make_async_copy(srcr_hbm.at[s, t1_], srcseg.at[slot],
                                  stsems[slot]).wait()
            pltpu.make_async_copy(dstr_hbm.at[s, t1_], dstseg.at[slot],
                                  stsems[slot]).wait()

        def run_seg(p):
            sseg = srcseg.at[p]
            dseg = dstseg.at[p]

            def fire_g(r, j):
                pltpu.async_copy(gsrc.at[sseg.at[r]], rows_v.at[j], gsems[j])

            def wait_g(r, j):
                pltpu.make_async_copy(gsrc.at[sseg.at[r]],
                                      rows_v.at[j], gsems[j]).wait()

            def fire_s(r, j):
                pltpu.async_copy(rows_v.at[j], acc_sh.at[dseg.at[r]],
                                 ssems[j], add=True)
                if with_deg:
                    @pl.when(c == 0)
                    def _():
                        pltpu.async_copy(ones_v, deg_sh.at[dseg.at[r]],
                                         dsem, add=True)

            def wait_s(r, j):
                pltpu.make_async_copy(rows_v.at[j], acc_sh.at[dseg.at[r]],
                                      ssems[j]).wait()

            fire_g(0, 0)
            fire_g(1, 1)
            fire_g(2, 2)

            def rr_body(rr, carry):
                base = 4 * rr

                @pl.when(rr > 0)
                def _():
                    wait_s(base - 1, 3)

                fire_g(base + 3, 3)
                wait_g(base, 0)
                fire_s(base, 0)
                wait_g(base + 1, 1)
                fire_s(base + 1, 1)
                wait_s(base, 0)

                @pl.when(rr < SEG // 4 - 1)
                def _():
                    fire_g(base + 4, 0)

                wait_g(base + 2, 2)
                fire_s(base + 2, 2)
                wait_s(base + 1, 1)

                @pl.when(rr < SEG // 4 - 1)
                def _():
                    fire_g(base + 5, 1)

                wait_g(base + 3, 3)
                fire_s(base + 3, 3)
                wait_s(base + 2, 2)

                @pl.when(rr < SEG // 4 - 1)
                def _():
                    fire_g(base + 6, 2)

                return carry

            lax.fori_loop(0, SEG // 4, rr_body, 0)
            wait_s(SEG - 1, 3)

        def t_body(t, carry):
            def go(p):
                @pl.when(t < NSEG - 1)
                def _():
                    fire_stage(t + 1, 1 - p)

                run_seg(p)

                @pl.when(t < NSEG - 1)
                def _():
                    wait_stage(t + 1, 1 - p)

            @pl.when(lax.rem(t, 2) == 0)
            def _():
                go(0)

            @pl.when(lax.rem(t, 2) == 1)
            def _():
                go(1)

            return carry

        lax.fori_loop(0, NSEG, t_body, 0)

        if with_deg:
            @pl.when(c == 0)
            def _():
                def d_body(k, carry):
                    pltpu.make_async_copy(ones_v, deg_sh.at[dstseg.at[0, 0]],
                                          dsem).wait()
                    return carry
                lax.fori_loop(0, KB, d_body, 0)

        plsc.subcore_barrier()
        sl = pl.ds(s * STRIPE, STRIPE)
        pltpu.sync_copy(acc_sh.at[sl], part_hbm.at[c, sl])
        if with_deg:
            dsl = pl.ds(s * DSTRIPE, DSTRIPE)

            @pl.when(c == 0)
            def _():
                pltpu.sync_copy(deg_sh.at[dsl], deg0_hbm.at[dsl])

            @pl.when(c == 1)
            def _():
                pltpu.sync_copy(deg_sh.at[dsl], deg1_hbm.at[dsl])

    return body


_sc_params = pltpu.CompilerParams(use_tc_tiling_on_sc=False)

_agg_sems = (
    pltpu.SemaphoreType.DMA,
    pltpu.SemaphoreType.DMA,
    pltpu.SemaphoreType.DMA,
    pltpu.SemaphoreType.DMA,
    pltpu.SemaphoreType.DMA,
    pltpu.SemaphoreType.DMA,
    pltpu.SemaphoreType.DMA,
    pltpu.SemaphoreType.DMA,
    pltpu.SemaphoreType.DMA,
    pltpu.SemaphoreType.DMA,
)

_sc_agg_deg = functools.partial(
    pl.kernel,
    mesh=_mesh,
    compiler_params=_sc_params,
    out_type=(
        jax.ShapeDtypeStruct((NC, NP, HD), jnp.float32),
        jax.ShapeDtypeStruct((NPD,), jnp.float32),
        jax.ShapeDtypeStruct((NPD,), jnp.float32),
    ),
    scratch_types=(
        pltpu.VMEM((2, SEG, B), jnp.int32),
        pltpu.VMEM((2, SEG, B), jnp.int32),
        pltpu.VMEM((4, B, HD), jnp.float32),
        pltpu.VMEM((B,), jnp.float32),
        pltpu.VMEM_SHARED((NP, HD), jnp.float32),
        pltpu.VMEM_SHARED((NPD,), jnp.float32),
    ) + _agg_sems + (pltpu.SemaphoreType.DMA,),
)(_sc_agg_body(True))


_sc_agg = functools.partial(
    pl.kernel,
    mesh=_mesh,
    compiler_params=_sc_params,
    out_type=jax.ShapeDtypeStruct((NC, NP, HD), jnp.float32),
    scratch_types=(
        pltpu.VMEM((2, SEG, B), jnp.int32),
        pltpu.VMEM((2, SEG, B), jnp.int32),
        pltpu.VMEM((4, B, HD), jnp.float32),
        pltpu.VMEM_SHARED((NP, HD), jnp.float32),
    ) + _agg_sems,
)(_sc_agg_body(False))


RB = 1000


def _mm_body(x_ref, w_ref, o_ref):
    o_ref[0] = jnp.dot(x_ref[...], w_ref[0],
                       preferred_element_type=jnp.float32)


def _tc_matmul(x, Wr):
    return pl.pallas_call(
        _mm_body,
        grid=(NC, N // RB),
        in_specs=[pl.BlockSpec((RB, D), lambda c, i: (i, 0)),
                  pl.BlockSpec((1, D, HD), lambda c, i: (c, 0, 0))],
        out_specs=pl.BlockSpec((1, RB, HD), lambda c, i: (c, i, 0)),
        out_shape=jax.ShapeDtypeStruct((NC, NP, HD), jnp.float32),
    )(x, Wr)


def _fused_body(g0_ref, g1_ref, p0_ref, p1_ref, d0_ref, d1_ref, b0_ref,
                b1_ref, w_ref, o_ref):
    inv = 1.0 / jnp.maximum(d0_ref[...] + d1_ref[...], 1.0)
    h0 = jnp.maximum(g0_ref[0] + b0_ref[0] + p0_ref[0] * inv, 0.0)
    h1 = jnp.maximum(g1_ref[0] + b1_ref[0] + p1_ref[0] * inv, 0.0)
    o_ref[0] = (jnp.dot(h0, w_ref[0, :HD, :],
                        preferred_element_type=jnp.float32)
                + jnp.dot(h1, w_ref[0, HD:, :],
                          preferred_element_type=jnp.float32))


def _tc_fused(g, p, d0, d1, br, Wr):
    return pl.pallas_call(
        _fused_body,
        grid=(NC, N // RB),
        in_specs=[pl.BlockSpec((1, RB, HD), lambda c, i: (0, i, 0)),
                  pl.BlockSpec((1, RB, HD), lambda c, i: (1, i, 0)),
                  pl.BlockSpec((1, RB, HD), lambda c, i: (0, i, 0)),
                  pl.BlockSpec((1, RB, HD), lambda c, i: (1, i, 0)),
                  pl.BlockSpec((RB, 1), lambda c, i: (i, 0)),
                  pl.BlockSpec((RB, 1), lambda c, i: (i, 0)),
                  pl.BlockSpec((1, 1, HD), lambda c, i: (0, 0, 0)),
                  pl.BlockSpec((1, 1, HD), lambda c, i: (1, 0, 0)),
                  pl.BlockSpec((1, D, HD), lambda c, i: (c, 0, 0))],
        out_specs=pl.BlockSpec((1, RB, HD), lambda c, i: (c, i, 0)),
        out_shape=jax.ShapeDtypeStruct((NC, NP, HD), jnp.float32),
    )(g, g, p, p, d0, d1, br, br, Wr)


def _final_body(g0_ref, g1_ref, p0_ref, p1_ref, d0_ref, d1_ref, b0_ref,
                b1_ref, o_ref):
    inv = 1.0 / jnp.maximum(d0_ref[...] + d1_ref[...], 1.0)
    o_ref[:, :HD] = g0_ref[0] + b0_ref[0] + p0_ref[0] * inv
    o_ref[:, HD:] = g1_ref[0] + b1_ref[0] + p1_ref[0] * inv


def _tc_final(g, p, d0, d1, br):
    return pl.pallas_call(
        _final_body,
        grid=(N // RB,),
        in_specs=[pl.BlockSpec((1, RB, HD), lambda i: (0, i, 0)),
                  pl.BlockSpec((1, RB, HD), lambda i: (1, i, 0)),
                  pl.BlockSpec((1, RB, HD), lambda i: (0, i, 0)),
                  pl.BlockSpec((1, RB, HD), lambda i: (1, i, 0)),
                  pl.BlockSpec((RB, 1), lambda i: (i, 0)),
                  pl.BlockSpec((RB, 1), lambda i: (i, 0)),
                  pl.BlockSpec((1, 1, HD), lambda i: (0, 0, 0)),
                  pl.BlockSpec((1, 1, HD), lambda i: (1, 0, 0))],
        out_specs=pl.BlockSpec((RB, D), lambda i: (i, 0)),
        out_shape=jax.ShapeDtypeStruct((N, D), jnp.float32),
    )(g, g, p, p, d0, d1, br, br)


def kernel(x, edge_index, W1, b1, W2, b2, W3, b3):
    npad = EPAD - E
    srcf = edge_index[0].astype(jnp.int32)
    dstf = edge_index[1].astype(jnp.int32)
    ar = jnp.arange(npad, dtype=jnp.int32)
    pad_src = (ar * 131) % N
    pad_dst = N + (ar % (NP - N))
    srcp = jnp.concatenate([srcf, pad_src]).reshape(NS, NSEG, SEG, B)
    dstp = jnp.concatenate([dstf, pad_dst]).reshape(NS, NSEG, SEG, B)
    z2d = jnp.zeros((STRIPE, HD), jnp.float32)
    z1d = jnp.zeros((DSTRIPE,), jnp.float32)
    ones = jnp.ones((B,), jnp.float32)
    W1r = jnp.stack((W1[:, :HD], W1[:, HD:]))
    W2r = jnp.stack((W2[:, :HD], W2[:, HD:]))
    W3r = jnp.stack((W3[:, :HD], W3[:, HD:]))
    b1r = b1.reshape(NC, 1, HD)
    b2r = b2.reshape(NC, 1, HD)
    b3r = b3.reshape(NC, 1, HD)

    g1 = _tc_matmul(x, W1r)
    p1, dg0, dg1 = _sc_agg_deg(g1, srcp, dstp, z2d, z1d, ones)
    d0 = dg0.reshape(NPD, 1)
    d1 = dg1.reshape(NPD, 1)
    g2 = _tc_fused(g1, p1, d0, d1, b1r, W2r)
    p2 = _sc_agg(g2, srcp, dstp, z2d)
    g3 = _tc_fused(g2, p2, d0, d1, b2r, W3r)
    p3 = _sc_agg(g3, srcp, dstp, z2d)
    return _tc_final(g3, p3, d0, d1, b3r)


# speedup vs baseline: 12.0290x; 1.1242x over previous
import functools

import jax
import jax.numpy as jnp
from jax import lax
from jax.experimental import pallas as pl
from jax.experimental.pallas import tpu as pltpu
from jax.experimental.pallas import tpu_sc as plsc

N = 10000
NP = 10112
NPD = 10240
E = 320000
EPAD = 327680
D = 128
NC = 2
NS = 16
NW = NC * NS
B = 64
KB = EPAD // (NW * B)
SEG = 16
NSEG = KB // SEG
STRIPE = NP // NS
DSTRIPE = NPD // NS

_mesh = plsc.VectorSubcoreMesh(core_axis_name="c", subcore_axis_name="s")


def _sc_agg_body(with_deg):
    def body(*args):
        if with_deg:
            (g_hbm, srcr_hbm, dstr_hbm, z2d_hbm, z1d_hbm, ones_hbm,
             part_hbm, deg0_hbm, deg1_hbm,
             srcseg, dstseg, rows_v, ones_v, acc_sh, deg_sh,
             g0, g1, g2, g3, s0, s1, s2, s3, t0, t1, dsem) = args
        else:
            (g_hbm, srcr_hbm, dstr_hbm, z2d_hbm,
             part_hbm,
             srcseg, dstseg, rows_v, acc_sh,
             g0, g1, g2, g3, s0, s1, s2, s3, t0, t1) = args
        gsems = (g0, g1, g2, g3)
        ssems = (s0, s1, s2, s3)
        stsems = (t0, t1)
        c = lax.axis_index("c")
        s = lax.axis_index("s")
        w = c * NS + s
        pltpu.sync_copy(srcr_hbm.at[w, 0], srcseg.at[0])
        pltpu.sync_copy(dstr_hbm.at[w, 0], dstseg.at[0])
        if with_deg:
            pltpu.sync_copy(ones_hbm, ones_v)
            pltpu.sync_copy(z1d_hbm, deg_sh.at[pl.ds(s * DSTRIPE, DSTRIPE)])
        pltpu.sync_copy(z2d_hbm, acc_sh.at[pl.ds(s * STRIPE, STRIPE)])
        plsc.subcore_barrier()

        def fire_stage(t1_, slot):
            pltpu.async_copy(srcr_hbm.at[w, t1_], srcseg.at[slot],
                             stsems[slot])
            pltpu.async_copy(dstr_hbm.at[w, t1_], dstseg.at[slot],
                             stsems[slot])

        def wait_stage(t1_, slot):
            pltpu.make_async_copy(srcr_hbm.at[w, t1_], srcseg.at[slot],
                                  stsems[slot]).wait()
            pltpu.make_async_copy(dstr_hbm.at[w, t1_], dstseg.at[slot],
                                  stsems[slot]).wait()

        def run_seg(p):
            sseg = srcseg.at[p]
            dseg = dstseg.at[p]

            def fire_g(r, j):
                pltpu.async_copy(g_hbm.at[sseg.at[r]], rows_v.at[j], gsems[j])

            def wait_g(r, j):
                pltpu.make_async_copy(g_hbm.at[sseg.at[r]],
                                      rows_v.at[j], gsems[j]).wait()

            def fire_s(r, j):
                pltpu.async_copy(rows_v.at[j], acc_sh.at[dseg.at[r]],
                                 ssems[j], add=True)
                if with_deg:
                    pltpu.async_copy(ones_v, deg_sh.at[dseg.at[r]],
                                     dsem, add=True)

            def wait_s(r, j):
                pltpu.make_async_copy(rows_v.at[j], acc_sh.at[dseg.at[r]],
                                      ssems[j]).wait()

            fire_g(0, 0)
            fire_g(1, 1)
            fire_g(2, 2)

            def rr_body(rr, carry):
                base = 4 * rr

                @pl.when(rr > 0)
                def _():
                    wait_s(base - 1, 3)

                fire_g(base + 3, 3)
                wait_g(base, 0)
                fire_s(base, 0)
                wait_g(base + 1, 1)
                fire_s(base + 1, 1)
                wait_s(base, 0)

                @pl.when(rr < SEG // 4 - 1)
                def _():
                    fire_g(base + 4, 0)

                wait_g(base + 2, 2)
                fire_s(base + 2, 2)
                wait_s(base + 1, 1)

                @pl.when(rr < SEG // 4 - 1)
                def _():
                    fire_g(base + 5, 1)

                wait_g(base + 3, 3)
                fire_s(base + 3, 3)
                wait_s(base + 2, 2)

                @pl.when(rr < SEG // 4 - 1)
                def _():
                    fire_g(base + 6, 2)

                return carry

            lax.fori_loop(0, SEG // 4, rr_body, 0)
            wait_s(SEG - 1, 3)

        def t_body(t, carry):
            def go(p):
                @pl.when(t < NSEG - 1)
                def _():
                    fire_stage(t + 1, 1 - p)

                run_seg(p)

                @pl.when(t < NSEG - 1)
                def _():
                    wait_stage(t + 1, 1 - p)

            @pl.when(lax.rem(t, 2) == 0)
            def _():
                go(0)

            @pl.when(lax.rem(t, 2) == 1)
            def _():
                go(1)

            return carry

        lax.fori_loop(0, NSEG, t_body, 0)

        if with_deg:
            def d_body(k, carry):
                pltpu.make_async_copy(ones_v, deg_sh.at[dstseg.at[0, 0]],
                                      dsem).wait()
                return carry
            lax.fori_loop(0, KB, d_body, 0)

        plsc.subcore_barrier()
        sl = pl.ds(s * STRIPE, STRIPE)
        pltpu.sync_copy(acc_sh.at[sl], part_hbm.at[c, sl])
        if with_deg:
            dsl = pl.ds(s * DSTRIPE, DSTRIPE)

            @pl.when(c == 0)
            def _():
                pltpu.sync_copy(deg_sh.at[dsl], deg0_hbm.at[dsl])

            @pl.when(c == 1)
            def _():
                pltpu.sync_copy(deg_sh.at[dsl], deg1_hbm.at[dsl])

    return body


_sc_params = pltpu.CompilerParams(use_tc_tiling_on_sc=False)

_agg_sems = (
    pltpu.SemaphoreType.DMA,
    pltpu.SemaphoreType.DMA,
    pltpu.SemaphoreType.DMA,
    pltpu.SemaphoreType.DMA,
    pltpu.SemaphoreType.DMA,
    pltpu.SemaphoreType.DMA,
    pltpu.SemaphoreType.DMA,
    pltpu.SemaphoreType.DMA,
    pltpu.SemaphoreType.DMA,
    pltpu.SemaphoreType.DMA,
)

_sc_agg_deg = functools.partial(
    pl.kernel,
    mesh=_mesh,
    compiler_params=_sc_params,
    out_type=(
        jax.ShapeDtypeStruct((NC, NP, D), jnp.float32),
        jax.ShapeDtypeStruct((NPD,), jnp.float32),
        jax.ShapeDtypeStruct((NPD,), jnp.float32),
    ),
    scratch_types=(
        pltpu.VMEM((2, SEG, B), jnp.int32),
        pltpu.VMEM((2, SEG, B), jnp.int32),
        pltpu.VMEM((4, B, D), jnp.float32),
        pltpu.VMEM((B,), jnp.float32),
        pltpu.VMEM_SHARED((NP, D), jnp.float32),
        pltpu.VMEM_SHARED((NPD,), jnp.float32),
    ) + _agg_sems + (pltpu.SemaphoreType.DMA,),
)(_sc_agg_body(True))


_sc_agg = functools.partial(
    pl.kernel,
    mesh=_mesh,
    compiler_params=_sc_params,
    out_type=jax.ShapeDtypeStruct((NC, NP, D), jnp.float32),
    scratch_types=(
        pltpu.VMEM((2, SEG, B), jnp.int32),
        pltpu.VMEM((2, SEG, B), jnp.int32),
        pltpu.VMEM((4, B, D), jnp.float32),
        pltpu.VMEM_SHARED((NP, D), jnp.float32),
    ) + _agg_sems,
)(_sc_agg_body(False))


RB = 1000


def _mm_body(x_ref, w_ref, o_ref):
    o_ref[...] = jnp.dot(x_ref[...], w_ref[...],
                         preferred_element_type=jnp.float32)


def _tc_matmul(x, W):
    return pl.pallas_call(
        _mm_body,
        grid=(N // RB,),
        in_specs=[pl.BlockSpec((RB, D), lambda i: (i, 0)),
                  pl.BlockSpec((D, D), lambda i: (0, 0))],
        out_specs=pl.BlockSpec((RB, D), lambda i: (i, 0)),
        out_shape=jax.ShapeDtypeStruct((N, D), jnp.float32),
    )(x, W)


def _fused_body(g_ref, p0_ref, p1_ref, d0_ref, d1_ref, b_ref, w_ref, o_ref):
    inv = 1.0 / jnp.maximum(d0_ref[...] + d1_ref[...], 1.0)
    h = g_ref[...] + b_ref[...] + (p0_ref[0] + p1_ref[0]) * inv
    h = jnp.maximum(h, 0.0)
    o_ref[...] = jnp.dot(h, w_ref[...], preferred_element_type=jnp.float32)


def _tc_fused(g, p, d0, d1, b, Wn):
    return pl.pallas_call(
        _fused_body,
        grid=(N // RB,),
        in_specs=[pl.BlockSpec((RB, D), lambda i: (i, 0)),
                  pl.BlockSpec((1, RB, D), lambda i: (0, i, 0)),
                  pl.BlockSpec((1, RB, D), lambda i: (1, i, 0)),
                  pl.BlockSpec((RB, 1), lambda i: (i, 0)),
                  pl.BlockSpec((RB, 1), lambda i: (i, 0)),
                  pl.BlockSpec((1, D), lambda i: (0, 0)),
                  pl.BlockSpec((D, D), lambda i: (0, 0))],
        out_specs=pl.BlockSpec((RB, D), lambda i: (i, 0)),
        out_shape=jax.ShapeDtypeStruct((N, D), jnp.float32),
    )(g, p, p, d0, d1, b, Wn)


def _final_body(g_ref, p0_ref, p1_ref, d0_ref, d1_ref, b_ref, o_ref):
    inv = 1.0 / jnp.maximum(d0_ref[...] + d1_ref[...], 1.0)
    o_ref[...] = (g_ref[...] + b_ref[...]
                  + (p0_ref[0] + p1_ref[0]) * inv)


def _tc_final(g, p, d0, d1, b):
    return pl.pallas_call(
        _final_body,
        grid=(N // RB,),
        in_specs=[pl.BlockSpec((RB, D), lambda i: (i, 0)),
                  pl.BlockSpec((1, RB, D), lambda i: (0, i, 0)),
                  pl.BlockSpec((1, RB, D), lambda i: (1, i, 0)),
                  pl.BlockSpec((RB, 1), lambda i: (i, 0)),
                  pl.BlockSpec((RB, 1), lambda i: (i, 0)),
                  pl.BlockSpec((1, D), lambda i: (0, 0))],
        out_specs=pl.BlockSpec((RB, D), lambda i: (i, 0)),
        out_shape=jax.ShapeDtypeStruct((N, D), jnp.float32),
    )(g, p, p, d0, d1, b)


def kernel(x, edge_index, W1, b1, W2, b2, W3, b3):
    npad = EPAD - E
    srcf = edge_index[0].astype(jnp.int32)
    dstf = edge_index[1].astype(jnp.int32)
    ar = jnp.arange(npad, dtype=jnp.int32)
    pad_src = (ar * 131) % N
    pad_dst = N + (ar % (NP - N))
    srcp = jnp.concatenate([srcf, pad_src]).reshape(NW, NSEG, SEG, B)
    dstp = jnp.concatenate([dstf, pad_dst]).reshape(NW, NSEG, SEG, B)
    z2d = jnp.zeros((STRIPE, D), jnp.float32)
    z1d = jnp.zeros((DSTRIPE,), jnp.float32)
    ones = jnp.ones((B,), jnp.float32)
    b1r = b1.reshape(1, D)
    b2r = b2.reshape(1, D)
    b3r = b3.reshape(1, D)

    g1 = _tc_matmul(x, W1)
    p1, dg0, dg1 = _sc_agg_deg(g1, srcp, dstp, z2d, z1d, ones)
    d0 = dg0.reshape(NPD, 1)
    d1 = dg1.reshape(NPD, 1)
    g2 = _tc_fused(g1, p1, d0, d1, b1r, W2)
    p2 = _sc_agg(g2, srcp, dstp, z2d)
    g3 = _tc_fused(g2, p2, d0, d1, b2r, W3)
    p3 = _sc_agg(g3, srcp, dstp, z2d)
    return _tc_final(g3, p3, d0, d1, b3r)


# speedup vs baseline: 12.1397x; 1.0092x over previous
import functools

import jax
import jax.numpy as jnp
from jax import lax
from jax.experimental import pallas as pl
from jax.experimental.pallas import tpu as pltpu
from jax.experimental.pallas import tpu_sc as plsc

N = 10000
NP = 10112
NPD = 10240
E = 320000
EPAD = 327680
D = 128
NC = 2
NS = 16
NW = NC * NS
B = 80
KB = EPAD // (NW * B)
SEG = 16
NSEG = KB // SEG
STRIPE = NP // NS
DSTRIPE = NPD // NS

_mesh = plsc.VectorSubcoreMesh(core_axis_name="c", subcore_axis_name="s")


def _sc_agg_body(with_deg):
    def body(*args):
        if with_deg:
            (g_hbm, srcr_hbm, dstr_hbm, z2d_hbm, z1d_hbm, ones_hbm,
             part_hbm, deg0_hbm, deg1_hbm,
             srcseg, dstseg, rows_v, ones_v, acc_sh, deg_sh,
             g0, g1, g2, g3, s0, s1, s2, s3, t0, t1, dsem) = args
        else:
            (g_hbm, srcr_hbm, dstr_hbm, z2d_hbm,
             part_hbm,
             srcseg, dstseg, rows_v, acc_sh,
             g0, g1, g2, g3, s0, s1, s2, s3, t0, t1) = args
        gsems = (g0, g1, g2, g3)
        ssems = (s0, s1, s2, s3)
        stsems = (t0, t1)
        c = lax.axis_index("c")
        s = lax.axis_index("s")
        w = c * NS + s
        pltpu.sync_copy(srcr_hbm.at[w, 0], srcseg.at[0])
        pltpu.sync_copy(dstr_hbm.at[w, 0], dstseg.at[0])
        if with_deg:
            pltpu.sync_copy(ones_hbm, ones_v)
            pltpu.sync_copy(z1d_hbm, deg_sh.at[pl.ds(s * DSTRIPE, DSTRIPE)])
        pltpu.sync_copy(z2d_hbm, acc_sh.at[pl.ds(s * STRIPE, STRIPE)])
        plsc.subcore_barrier()

        def fire_stage(t1_, slot):
            pltpu.async_copy(srcr_hbm.at[w, t1_], srcseg.at[slot],
                             stsems[slot])
            pltpu.async_copy(dstr_hbm.at[w, t1_], dstseg.at[slot],
                             stsems[slot])

        def wait_stage(t1_, slot):
            pltpu.make_async_copy(srcr_hbm.at[w, t1_], srcseg.at[slot],
                                  stsems[slot]).wait()
            pltpu.make_async_copy(dstr_hbm.at[w, t1_], dstseg.at[slot],
                                  stsems[slot]).wait()

        def run_seg(p):
            sseg = srcseg.at[p]
            dseg = dstseg.at[p]

            def fire_g(r, j):
                pltpu.async_copy(g_hbm.at[sseg.at[r]], rows_v.at[j], gsems[j])

            def wait_g(r, j):
                pltpu.make_async_copy(g_hbm.at[sseg.at[r]],
                                      rows_v.at[j], gsems[j]).wait()

            def fire_s(r, j):
                pltpu.async_copy(rows_v.at[j], acc_sh.at[dseg.at[r]],
                                 ssems[j], add=True)
                if with_deg:
                    pltpu.async_copy(ones_v, deg_sh.at[dseg.at[r]],
                                     dsem, add=True)

            def wait_s(r, j):
                pltpu.make_async_copy(rows_v.at[j], acc_sh.at[dseg.at[r]],
                                      ssems[j]).wait()

            fire_g(0, 0)
            fire_g(1, 1)
            fire_g(2, 2)

            def rr_body(rr, carry):
                base = 4 * rr

                @pl.when(rr > 0)
                def _():
                    wait_s(base - 1, 3)

                fire_g(base + 3, 3)
                wait_g(base, 0)
                fire_s(base, 0)
                wait_g(base + 1, 1)
                fire_s(base + 1, 1)
                wait_s(base, 0)

                @pl.when(rr < SEG // 4 - 1)
                def _():
                    fire_g(base + 4, 0)

                wait_g(base + 2, 2)
                fire_s(base + 2, 2)
                wait_s(base + 1, 1)

                @pl.when(rr < SEG // 4 - 1)
                def _():
                    fire_g(base + 5, 1)

                wait_g(base + 3, 3)
                fire_s(base + 3, 3)
                wait_s(base + 2, 2)

                @pl.when(rr < SEG // 4 - 1)
                def _():
                    fire_g(base + 6, 2)

                return carry

            lax.fori_loop(0, SEG // 4, rr_body, 0)
            wait_s(SEG - 1, 3)

        def t_body(t, carry):
            def go(p):
                @pl.when(t < NSEG - 1)
                def _():
                    fire_stage(t + 1, 1 - p)

                run_seg(p)

                @pl.when(t < NSEG - 1)
                def _():
                    wait_stage(t + 1, 1 - p)

            @pl.when(lax.rem(t, 2) == 0)
            def _():
                go(0)

            @pl.when(lax.rem(t, 2) == 1)
            def _():
                go(1)

            return carry

        lax.fori_loop(0, NSEG, t_body, 0)

        if with_deg:
            def d_body(k, carry):
                pltpu.make_async_copy(ones_v, deg_sh.at[dstseg.at[0, 0]],
                                      dsem).wait()
                return carry
            lax.fori_loop(0, KB, d_body, 0)

        plsc.subcore_barrier()
        sl = pl.ds(s * STRIPE, STRIPE)
        pltpu.sync_copy(acc_sh.at[sl], part_hbm.at[c, sl])
        if with_deg:
            dsl = pl.ds(s * DSTRIPE, DSTRIPE)

            @pl.when(c == 0)
            def _():
                pltpu.sync_copy(deg_sh.at[dsl], deg0_hbm.at[dsl])

            @pl.when(c == 1)
            def _():
                pltpu.sync_copy(deg_sh.at[dsl], deg1_hbm.at[dsl])

    return body


_sc_params = pltpu.CompilerParams(use_tc_tiling_on_sc=False)

_agg_sems = (
    pltpu.SemaphoreType.DMA,
    pltpu.SemaphoreType.DMA,
    pltpu.SemaphoreType.DMA,
    pltpu.SemaphoreType.DMA,
    pltpu.SemaphoreType.DMA,
    pltpu.SemaphoreType.DMA,
    pltpu.SemaphoreType.DMA,
    pltpu.SemaphoreType.DMA,
    pltpu.SemaphoreType.DMA,
    pltpu.SemaphoreType.DMA,
)

_sc_agg_deg = functools.partial(
    pl.kernel,
    mesh=_mesh,
    compiler_params=_sc_params,
    out_type=(
        jax.ShapeDtypeStruct((NC, NP, D), jnp.float32),
        jax.ShapeDtypeStruct((NPD,), jnp.float32),
        jax.ShapeDtypeStruct((NPD,), jnp.float32),
    ),
    scratch_types=(
        pltpu.VMEM((2, SEG, B), jnp.int32),
        pltpu.VMEM((2, SEG, B), jnp.int32),
        pltpu.VMEM((4, B, D), jnp.float32),
        pltpu.VMEM((B,), jnp.float32),
        pltpu.VMEM_SHARED((NP, D), jnp.float32),
        pltpu.VMEM_SHARED((NPD,), jnp.float32),
    ) + _agg_sems + (pltpu.SemaphoreType.DMA,),
)(_sc_agg_body(True))


_sc_agg = functools.partial(
    pl.kernel,
    mesh=_mesh,
    compiler_params=_sc_params,
    out_type=jax.ShapeDtypeStruct((NC, NP, D), jnp.float32),
    scratch_types=(
        pltpu.VMEM((2, SEG, B), jnp.int32),
        pltpu.VMEM((2, SEG, B), jnp.int32),
        pltpu.VMEM((4, B, D), jnp.float32),
        pltpu.VMEM_SHARED((NP, D), jnp.float32),
    ) + _agg_sems,
)(_sc_agg_body(False))


RB = 1000


def _mm_body(x_ref, w_ref, o_ref):
    o_ref[...] = jnp.dot(x_ref[...], w_ref[...],
                         preferred_element_type=jnp.float32)


def _tc_matmul(x, W):
    return pl.pallas_call(
        _mm_body,
        grid=(N // RB,),
        in_specs=[pl.BlockSpec((RB, D), lambda i: (i, 0)),
                  pl.BlockSpec((D, D), lambda i: (0, 0))],
        out_specs=pl.BlockSpec((RB, D), lambda i: (i, 0)),
        out_shape=jax.ShapeDtypeStruct((N, D), jnp.float32),
    )(x, W)


def _fused_body(g_ref, p0_ref, p1_ref, d0_ref, d1_ref, b_ref, w_ref, o_ref):
    inv = 1.0 / jnp.maximum(d0_ref[...] + d1_ref[...], 1.0)
    h = g_ref[...] + b_ref[...] + (p0_ref[0] + p1_ref[0]) * inv
    h = jnp.maximum(h, 0.0)
    o_ref[...] = jnp.dot(h, w_ref[...], preferred_element_type=jnp.float32)


def _tc_fused(g, p, d0, d1, b, Wn):
    return pl.pallas_call(
        _fused_body,
        grid=(N // RB,),
        in_specs=[pl.BlockSpec((RB, D), lambda i: (i, 0)),
                  pl.BlockSpec((1, RB, D), lambda i: (0, i, 0)),
                  pl.BlockSpec((1, RB, D), lambda i: (1, i, 0)),
                  pl.BlockSpec((RB, 1), lambda i: (i, 0)),
                  pl.BlockSpec((RB, 1), lambda i: (i, 0)),
                  pl.BlockSpec((1, D), lambda i: (0, 0)),
                  pl.BlockSpec((D, D), lambda i: (0, 0))],
        out_specs=pl.BlockSpec((RB, D), lambda i: (i, 0)),
        out_shape=jax.ShapeDtypeStruct((N, D), jnp.float32),
    )(g, p, p, d0, d1, b, Wn)


def _final_body(g_ref, p0_ref, p1_ref, d0_ref, d1_ref, b_ref, o_ref):
    inv = 1.0 / jnp.maximum(d0_ref[...] + d1_ref[...], 1.0)
    o_ref[...] = (g_ref[...] + b_ref[...]
                  + (p0_ref[0] + p1_ref[0]) * inv)


def _tc_final(g, p, d0, d1, b):
    return pl.pallas_call(
        _final_body,
        grid=(N // RB,),
        in_specs=[pl.BlockSpec((RB, D), lambda i: (i, 0)),
                  pl.BlockSpec((1, RB, D), lambda i: (0, i, 0)),
                  pl.BlockSpec((1, RB, D), lambda i: (1, i, 0)),
                  pl.BlockSpec((RB, 1), lambda i: (i, 0)),
                  pl.BlockSpec((RB, 1), lambda i: (i, 0)),
                  pl.BlockSpec((1, D), lambda i: (0, 0))],
        out_specs=pl.BlockSpec((RB, D), lambda i: (i, 0)),
        out_shape=jax.ShapeDtypeStruct((N, D), jnp.float32),
    )(g, p, p, d0, d1, b)


def kernel(x, edge_index, W1, b1, W2, b2, W3, b3):
    npad = EPAD - E
    srcf = edge_index[0].astype(jnp.int32)
    dstf = edge_index[1].astype(jnp.int32)
    ar = jnp.arange(npad, dtype=jnp.int32)
    pad_src = (ar * 131) % N
    pad_dst = N + (ar % (NP - N))
    srcp = jnp.concatenate([srcf, pad_src]).reshape(NW, NSEG, SEG, B)
    dstp = jnp.concatenate([dstf, pad_dst]).reshape(NW, NSEG, SEG, B)
    z2d = jnp.zeros((STRIPE, D), jnp.float32)
    z1d = jnp.zeros((DSTRIPE,), jnp.float32)
    ones = jnp.ones((B,), jnp.float32)
    b1r = b1.reshape(1, D)
    b2r = b2.reshape(1, D)
    b3r = b3.reshape(1, D)

    g1 = _tc_matmul(x, W1)
    p1, dg0, dg1 = _sc_agg_deg(g1, srcp, dstp, z2d, z1d, ones)
    d0 = dg0.reshape(NPD, 1)
    d1 = dg1.reshape(NPD, 1)
    g2 = _tc_fused(g1, p1, d0, d1, b1r, W2)
    p2 = _sc_agg(g2, srcp, dstp, z2d)
    g3 = _tc_fused(g2, p2, d0, d1, b2r, W3)
    p3 = _sc_agg(g3, srcp, dstp, z2d)
    return _tc_final(g3, p3, d0, d1, b3r)


# speedup vs baseline: 12.1510x; 1.0009x over previous
import functools

import jax
import jax.numpy as jnp
from jax import lax
from jax.experimental import pallas as pl
from jax.experimental.pallas import tpu as pltpu
from jax.experimental.pallas import tpu_sc as plsc

N = 10000
NP = 10112
NPD = 10240
E = 320000
EPAD = 327680
D = 128
NC = 2
NS = 16
NW = NC * NS
B = 80
KB = EPAD // (NW * B)
SEG = 16
NSEG = KB // SEG
STRIPE = NP // NS
DSTRIPE = NPD // NS

_mesh = plsc.VectorSubcoreMesh(core_axis_name="c", subcore_axis_name="s")


def _sc_agg_body(with_deg):
    def body(*args):
        if with_deg:
            (g_hbm, srcr_hbm, dstr_hbm, z2d_hbm, z1d_hbm, ones_hbm,
             part_hbm, deg0_hbm, deg1_hbm,
             srcseg, dstseg, rows_v, ones_v, acc_sh, deg_sh,
             g0, g1, g2, g3, s0, s1, s2, s3, t0, t1, dsem) = args
        else:
            (g_hbm, srcr_hbm, dstr_hbm, z2d_hbm,
             part_hbm,
             srcseg, dstseg, rows_v, acc_sh,
             g0, g1, g2, g3, s0, s1, s2, s3, t0, t1) = args
        gsems = (g0, g1, g2, g3)
        ssems = (s0, s1, s2, s3)
        stsems = (t0, t1)
        c = lax.axis_index("c")
        s = lax.axis_index("s")
        w = c * NS + s
        pltpu.sync_copy(srcr_hbm.at[w, 0], srcseg.at[0])
        pltpu.sync_copy(dstr_hbm.at[w, 0], dstseg.at[0])
        if with_deg:
            pltpu.sync_copy(ones_hbm, ones_v)
            pltpu.sync_copy(z1d_hbm.at[pl.ds(s * DSTRIPE, DSTRIPE)],
                            deg_sh.at[pl.ds(s * DSTRIPE, DSTRIPE)])
        pltpu.sync_copy(z2d_hbm.at[pl.ds(s * STRIPE, STRIPE)],
                        acc_sh.at[pl.ds(s * STRIPE, STRIPE)])
        plsc.subcore_barrier()

        def fire_stage(t1_, slot):
            pltpu.async_copy(srcr_hbm.at[w, t1_], srcseg.at[slot],
                             stsems[slot])
            pltpu.async_copy(dstr_hbm.at[w, t1_], dstseg.at[slot],
                             stsems[slot])

        def wait_stage(t1_, slot):
            pltpu.make_async_copy(srcr_hbm.at[w, t1_], srcseg.at[slot],
                                  stsems[slot]).wait()
            pltpu.make_async_copy(dstr_hbm.at[w, t1_], dstseg.at[slot],
                                  stsems[slot]).wait()

        def run_seg(p):
            sseg = srcseg.at[p]
            dseg = dstseg.at[p]

            def fire_g(r, j):
                pltpu.async_copy(g_hbm.at[sseg.at[r]], rows_v.at[j], gsems[j])

            def wait_g(r, j):
                pltpu.make_async_copy(g_hbm.at[sseg.at[r]],
                                      rows_v.at[j], gsems[j]).wait()

            def fire_s(r, j):
                pltpu.async_copy(rows_v.at[j], acc_sh.at[dseg.at[r]],
                                 ssems[j], add=True)
                if with_deg:
                    pltpu.async_copy(ones_v, deg_sh.at[dseg.at[r]],
                                     dsem, add=True)

            def wait_s(r, j):
                pltpu.make_async_copy(rows_v.at[j], acc_sh.at[dseg.at[r]],
                                      ssems[j]).wait()

            fire_g(0, 0)
            fire_g(1, 1)
            fire_g(2, 2)

            def rr_body(rr, carry):
                base = 4 * rr

                @pl.when(rr > 0)
                def _():
                    wait_s(base - 1, 3)

                fire_g(base + 3, 3)
                wait_g(base, 0)
                fire_s(base, 0)
                wait_g(base + 1, 1)
                fire_s(base + 1, 1)
                wait_s(base, 0)

                @pl.when(rr < SEG // 4 - 1)
                def _():
                    fire_g(base + 4, 0)

                wait_g(base + 2, 2)
                fire_s(base + 2, 2)
                wait_s(base + 1, 1)

                @pl.when(rr < SEG // 4 - 1)
                def _():
                    fire_g(base + 5, 1)

                wait_g(base + 3, 3)
                fire_s(base + 3, 3)
                wait_s(base + 2, 2)

                @pl.when(rr < SEG // 4 - 1)
                def _():
                    fire_g(base + 6, 2)

                return carry

            lax.fori_loop(0, SEG // 4, rr_body, 0)
            wait_s(SEG - 1, 3)

        def t_body(t, carry):
            def go(p):
                @pl.when(t < NSEG - 1)
                def _():
                    fire_stage(t + 1, 1 - p)

                run_seg(p)

                @pl.when(t < NSEG - 1)
                def _():
                    wait_stage(t + 1, 1 - p)

            @pl.when(lax.rem(t, 2) == 0)
            def _():
                go(0)

            @pl.when(lax.rem(t, 2) == 1)
            def _():
                go(1)

            return carry

        lax.fori_loop(0, NSEG, t_body, 0)

        if with_deg:
            def d_body(k, carry):
                pltpu.make_async_copy(ones_v, deg_sh.at[dstseg.at[0, 0]],
                                      dsem).wait()
                return carry
            lax.fori_loop(0, KB, d_body, 0)

        plsc.subcore_barrier()
        sl = pl.ds(s * STRIPE, STRIPE)
        pltpu.sync_copy(acc_sh.at[sl], part_hbm.at[c, sl])
        if with_deg:
            dsl = pl.ds(s * DSTRIPE, DSTRIPE)

            @pl.when(c == 0)
            def _():
                pltpu.sync_copy(deg_sh.at[dsl], deg0_hbm.at[dsl])

            @pl.when(c == 1)
            def _():
                pltpu.sync_copy(deg_sh.at[dsl], deg1_hbm.at[dsl])

    return body


_sc_params = pltpu.CompilerParams(use_tc_tiling_on_sc=False)

_agg_sems = (
    pltpu.SemaphoreType.DMA,
    pltpu.SemaphoreType.DMA,
    pltpu.SemaphoreType.DMA,
    pltpu.SemaphoreType.DMA,
    pltpu.SemaphoreType.DMA,
    pltpu.SemaphoreType.DMA,
    pltpu.SemaphoreType.DMA,
    pltpu.SemaphoreType.DMA,
    pltpu.SemaphoreType.DMA,
    pltpu.SemaphoreType.DMA,
)

_sc_agg_deg = functools.partial(
    pl.kernel,
    mesh=_mesh,
    compiler_params=_sc_params,
    out_type=(
        jax.ShapeDtypeStruct((NC, NP, D), jnp.float32),
        jax.ShapeDtypeStruct((NPD,), jnp.float32),
        jax.ShapeDtypeStruct((NPD,), jnp.float32),
    ),
    scratch_types=(
        pltpu.VMEM((2, SEG, B), jnp.int32),
        pltpu.VMEM((2, SEG, B), jnp.int32),
        pltpu.VMEM((4, B, D), jnp.float32),
        pltpu.VMEM((B,), jnp.float32),
        pltpu.VMEM_SHARED((NP, D), jnp.float32),
        pltpu.VMEM_SHARED((NPD,), jnp.float32),
    ) + _agg_sems + (pltpu.SemaphoreType.DMA,),
)(_sc_agg_body(True))


_sc_agg = functools.partial(
    pl.kernel,
    mesh=_mesh,
    compiler_params=_sc_params,
    out_type=jax.ShapeDtypeStruct((NC, NP, D), jnp.float32),
    scratch_types=(
        pltpu.VMEM((2, SEG, B), jnp.int32),
        pltpu.VMEM((2, SEG, B), jnp.int32),
        pltpu.VMEM((4, B, D), jnp.float32),
        pltpu.VMEM_SHARED((NP, D), jnp.float32),
    ) + _agg_sems,
)(_sc_agg_body(False))


RB = 1000


def _mm_body(x_ref, w_ref, o_ref):
    o_ref[...] = jnp.dot(x_ref[...], w_ref[...],
                         preferred_element_type=jnp.float32)


def _tc_matmul(x, W):
    return pl.pallas_call(
        _mm_body,
        grid=(N // RB,),
        in_specs=[pl.BlockSpec((RB, D), lambda i: (i, 0)),
                  pl.BlockSpec((D, D), lambda i: (0, 0))],
        out_specs=pl.BlockSpec((RB, D), lambda i: (i, 0)),
        out_shape=jax.ShapeDtypeStruct((N, D), jnp.float32),
    )(x, W)


def _fused_body(g_ref, p0_ref, p1_ref, d0_ref, d1_ref, b_ref, w_ref, o_ref):
    inv = 1.0 / jnp.maximum(d0_ref[...] + d1_ref[...], 1.0)
    h = g_ref[...] + b_ref[...] + (p0_ref[0] + p1_ref[0]) * inv
    h = jnp.maximum(h, 0.0)
    o_ref[...] = jnp.dot(h, w_ref[...], preferred_element_type=jnp.float32)


def _tc_fused(g, p, d0, d1, b, Wn):
    return pl.pallas_call(
        _fused_body,
        grid=(N // RB,),
        in_specs=[pl.BlockSpec((RB, D), lambda i: (i, 0)),
                  pl.BlockSpec((1, RB, D), lambda i: (0, i, 0)),
                  pl.BlockSpec((1, RB, D), lambda i: (1, i, 0)),
                  pl.BlockSpec((RB, 1), lambda i: (i, 0)),
                  pl.BlockSpec((RB, 1), lambda i: (i, 0)),
                  pl.BlockSpec((1, D), lambda i: (0, 0)),
                  pl.BlockSpec((D, D), lambda i: (0, 0))],
        out_specs=pl.BlockSpec((RB, D), lambda i: (i, 0)),
        out_shape=jax.ShapeDtypeStruct((N, D), jnp.float32),
    )(g, p, p, d0, d1, b, Wn)


def _final_body(g_ref, p0_ref, p1_ref, d0_ref, d1_ref, b_ref, o_ref):
    inv = 1.0 / jnp.maximum(d0_ref[...] + d1_ref[...], 1.0)
    o_ref[...] = (g_ref[...] + b_ref[...]
                  + (p0_ref[0] + p1_ref[0]) * inv)


def _tc_final(g, p, d0, d1, b):
    return pl.pallas_call(
        _final_body,
        grid=(N // RB,),
        in_specs=[pl.BlockSpec((RB, D), lambda i: (i, 0)),
                  pl.BlockSpec((1, RB, D), lambda i: (0, i, 0)),
                  pl.BlockSpec((1, RB, D), lambda i: (1, i, 0)),
                  pl.BlockSpec((RB, 1), lambda i: (i, 0)),
                  pl.BlockSpec((RB, 1), lambda i: (i, 0)),
                  pl.BlockSpec((1, D), lambda i: (0, 0))],
        out_specs=pl.BlockSpec((RB, D), lambda i: (i, 0)),
        out_shape=jax.ShapeDtypeStruct((N, D), jnp.float32),
    )(g, p, p, d0, d1, b)


def kernel(x, edge_index, W1, b1, W2, b2, W3, b3):
    npad = EPAD - E
    srcf = edge_index[0].astype(jnp.int32)
    dstf = edge_index[1].astype(jnp.int32)
    ar = jnp.arange(npad, dtype=jnp.int32)
    pad_src = (ar * 131) % N
    pad_dst = N + (ar % (NP - N))
    srcp = jnp.concatenate([srcf, pad_src]).reshape(NW, NSEG, SEG, B)
    dstp = jnp.concatenate([dstf, pad_dst]).reshape(NW, NSEG, SEG, B)
    z2d = jnp.zeros((NP, D), jnp.float32)
    z1d = jnp.zeros((NPD,), jnp.float32)
    ones = jnp.ones((B,), jnp.float32)
    b1r = b1.reshape(1, D)
    b2r = b2.reshape(1, D)
    b3r = b3.reshape(1, D)

    g1 = _tc_matmul(x, W1)
    p1, dg0, dg1 = _sc_agg_deg(g1, srcp, dstp, z2d, z1d, ones)
    d0 = dg0.reshape(NPD, 1)
    d1 = dg1.reshape(NPD, 1)
    g2 = _tc_fused(g1, p1, d0, d1, b1r, W2)
    p2 = _sc_agg(g2, srcp, dstp, z2d)
    g3 = _tc_fused(g2, p2, d0, d1, b2r, W3)
    p3 = _sc_agg(g3, srcp, dstp, z2d)
    return _tc_final(g3, p3, d0, d1, b3r)
